# baseline (speedup 1.0000x reference)
import functools

import jax
import jax.numpy as jnp
from jax.experimental import pallas as pl
from jax.experimental.pallas import tpu as pltpu

B = 16384
D = 512
Q = 4
K = 1024
H = 512
OUT = 7

BM = 512


def _body(z_ref, cb_cat_ref, cbT_ref, W1_ref, b1_ref,
          W2_ref, b2_ref, W3_ref, b3_ref, out_ref, qscratch_ref):
    z = z_ref[...]
    r = z
    acc = jnp.zeros_like(z)
    for q in range(Q):
        cbT = cbT_ref[q]
        cn = jnp.sum(cbT * cbT, axis=0, keepdims=True)
        rn = jnp.sum(r * r, axis=1, keepdims=True)
        s = jnp.dot(r.astype(jnp.bfloat16), cbT.astype(jnp.bfloat16),
                    preferred_element_type=jnp.float32)
        d2 = rn - 2.0 * s + cn
        m = jnp.min(d2, axis=1, keepdims=True)
        iota = jax.lax.broadcasted_iota(jnp.int32, d2.shape, 1)
        idx = jnp.min(jnp.where(d2 == m, iota, K), axis=1, keepdims=True)
        onehot = (iota == idx).astype(jnp.bfloat16)
        oh3 = jnp.concatenate([onehot, onehot, onehot], axis=1)
        qscratch_ref[...] = jnp.dot(oh3, cb_cat_ref[q],
                                    preferred_element_type=jnp.float32)
        quant = qscratch_ref[...]
        r = r - quant
        acc = acc + quant
    dec_in = z + (acc - z)
    h = jnp.maximum(
        jnp.dot(dec_in.astype(jnp.bfloat16), W1_ref[...].astype(jnp.bfloat16),
                preferred_element_type=jnp.float32)
        + b1_ref[...], 0.0)
    h = jnp.maximum(
        jnp.dot(h.astype(jnp.bfloat16), W2_ref[...].astype(jnp.bfloat16),
                preferred_element_type=jnp.float32)
        + b2_ref[...], 0.0)
    out_ref[...] = (
        jnp.dot(h.astype(jnp.bfloat16), W3_ref[...].astype(jnp.bfloat16),
                preferred_element_type=jnp.float32)
        + b3_ref[...])


@jax.jit
def kernel(z, codebooks, W1, b1, W2, b2, W3, b3):
    cbT = codebooks.transpose(0, 2, 1)
    cb_hi = codebooks.astype(jnp.bfloat16)
    hi_f = jax.lax.optimization_barrier(cb_hi).astype(jnp.float32)
    rem = codebooks - hi_f
    cb_mid = rem.astype(jnp.bfloat16)
    mid_f = jax.lax.optimization_barrier(cb_mid).astype(jnp.float32)
    cb_lo = (rem - mid_f).astype(jnp.bfloat16)
    cb_cat = jnp.concatenate([cb_hi, cb_mid, cb_lo], axis=1)
    b1 = b1.reshape(1, H)
    b2 = b2.reshape(1, H)
    b3 = b3.reshape(1, OUT)
    grid = (B // BM,)

    def c2(i):
        return (0, 0)

    def c3(i):
        return (0, 0, 0)

    return pl.pallas_call(
        _body,
        grid=grid,
        in_specs=[
            pl.BlockSpec((BM, D), lambda i: (i, 0)),
            pl.BlockSpec((Q, 3 * K, D), c3),
            pl.BlockSpec((Q, D, K), c3),
            pl.BlockSpec((D, H), c2),
            pl.BlockSpec((1, H), c2),
            pl.BlockSpec((H, H), c2),
            pl.BlockSpec((1, H), c2),
            pl.BlockSpec((H, OUT), c2),
            pl.BlockSpec((1, OUT), c2),
        ],
        out_specs=pl.BlockSpec((BM, OUT), lambda i: (i, 0)),
        scratch_shapes=[pltpu.VMEM((BM, D), jnp.float32)],
        out_shape=jax.ShapeDtypeStruct((B, OUT), jnp.float32),
        compiler_params=pltpu.CompilerParams(
            dimension_semantics=("parallel",),
        ),
    )(z, cb_cat, cbT, W1, b1, W2, b2, W3, b3)


# speedup vs baseline: 1.0543x; 1.0543x over previous
import functools

import jax
import jax.numpy as jnp
from jax.experimental import pallas as pl
from jax.experimental.pallas import tpu as pltpu

B = 16384
D = 512
Q = 4
K = 1024
H = 512
OUT = 7

BM = 1024


def _body(z_ref, cb_cat_ref, cbT_ref, W1_ref, b1_ref,
          W2_ref, b2_ref, W3_ref, b3_ref, out_ref, qscratch_ref):
    z = z_ref[...]
    r = z
    acc = jnp.zeros_like(z)
    for q in range(Q):
        cbT = cbT_ref[q]
        cn = jnp.sum(cbT * cbT, axis=0, keepdims=True)
        rn = jnp.sum(r * r, axis=1, keepdims=True)
        s = jnp.dot(r.astype(jnp.bfloat16), cbT.astype(jnp.bfloat16),
                    preferred_element_type=jnp.float32)
        d2 = rn - 2.0 * s + cn
        m = jnp.min(d2, axis=1, keepdims=True)
        iota = jax.lax.broadcasted_iota(jnp.int32, d2.shape, 1)
        idx = jnp.min(jnp.where(d2 == m, iota, K), axis=1, keepdims=True)
        onehot = (iota == idx).astype(jnp.bfloat16)
        oh3 = jnp.concatenate([onehot, onehot, onehot], axis=1)
        qscratch_ref[...] = jnp.dot(oh3, cb_cat_ref[q],
                                    preferred_element_type=jnp.float32)
        quant = qscratch_ref[...]
        r = r - quant
        acc = acc + quant
    dec_in = z + (acc - z)
    h = jnp.maximum(
        jnp.dot(dec_in.astype(jnp.bfloat16), W1_ref[...].astype(jnp.bfloat16),
                preferred_element_type=jnp.float32)
        + b1_ref[...], 0.0)
    h = jnp.maximum(
        jnp.dot(h.astype(jnp.bfloat16), W2_ref[...].astype(jnp.bfloat16),
                preferred_element_type=jnp.float32)
        + b2_ref[...], 0.0)
    out_ref[...] = (
        jnp.dot(h.astype(jnp.bfloat16), W3_ref[...].astype(jnp.bfloat16),
                preferred_element_type=jnp.float32)
        + b3_ref[...])


@jax.jit
def kernel(z, codebooks, W1, b1, W2, b2, W3, b3):
    cbT = codebooks.transpose(0, 2, 1)
    cb_hi = codebooks.astype(jnp.bfloat16)
    hi_f = jax.lax.optimization_barrier(cb_hi).astype(jnp.float32)
    rem = codebooks - hi_f
    cb_mid = rem.astype(jnp.bfloat16)
    mid_f = jax.lax.optimization_barrier(cb_mid).astype(jnp.float32)
    cb_lo = (rem - mid_f).astype(jnp.bfloat16)
    cb_cat = jnp.concatenate([cb_hi, cb_mid, cb_lo], axis=1)
    b1 = b1.reshape(1, H)
    b2 = b2.reshape(1, H)
    b3 = b3.reshape(1, OUT)
    grid = (B // BM,)

    def c2(i):
        return (0, 0)

    def c3(i):
        return (0, 0, 0)

    return pl.pallas_call(
        _body,
        grid=grid,
        in_specs=[
            pl.BlockSpec((BM, D), lambda i: (i, 0)),
            pl.BlockSpec((Q, 3 * K, D), c3),
            pl.BlockSpec((Q, D, K), c3),
            pl.BlockSpec((D, H), c2),
            pl.BlockSpec((1, H), c2),
            pl.BlockSpec((H, H), c2),
            pl.BlockSpec((1, H), c2),
            pl.BlockSpec((H, OUT), c2),
            pl.BlockSpec((1, OUT), c2),
        ],
        out_specs=pl.BlockSpec((BM, OUT), lambda i: (i, 0)),
        scratch_shapes=[pltpu.VMEM((BM, D), jnp.float32)],
        out_shape=jax.ShapeDtypeStruct((B, OUT), jnp.float32),
        compiler_params=pltpu.CompilerParams(
            dimension_semantics=("parallel",),
        ),
    )(z, cb_cat, cbT, W1, b1, W2, b2, W3, b3)
